# baseline (speedup 1.0000x reference)
import jax
import jax.numpy as jnp
from jax import lax
from jax.experimental import pallas as pl
from jax.experimental.pallas import tpu as pltpu

N_FIELDS = 26
N_COLS = N_FIELDS + 2
FIELD_VOCAB = 100000
VEC_DIM = 16
BATCH = 16384
TOTAL_DIM = N_COLS * VEC_DIM

_BB = 8192
_NB = BATCH // _BB


def _fm_body(*refs):
    v_refs = refs[:N_COLS]
    k_ref, w_ref, b_ref, o_ref = refs[N_COLS:]
    v = jnp.concatenate([r[...] for r in v_refs], axis=0)
    k = k_ref[...]
    w = w_ref[...]
    dn = (((0,), (0,)), ((), ()))
    s = lax.dot_general(k, v, dn, preferred_element_type=jnp.float32)
    q = lax.dot_general(k * k, v * v, dn, preferred_element_type=jnp.float32)
    liner = lax.dot_general(w, v, dn, preferred_element_type=jnp.float32)
    cross = 0.5 * jnp.sum(s * s - q, axis=0, keepdims=True)
    z = liner + b_ref[0, 0] + cross
    o_ref[...] = 1.0 / (1.0 + jnp.exp(-z))


def _fm(user_t, item_t, feat_t, k_mat, w, b2):
    in_specs = [
        pl.BlockSpec((VEC_DIM, _BB), lambda i: (0, i)),
        pl.BlockSpec((VEC_DIM, _BB), lambda i: (0, i)),
    ]
    for f in range(N_FIELDS):
        in_specs.append(
            pl.BlockSpec((VEC_DIM, _BB), lambda i, f=f: (0, f * _NB + i))
        )
    in_specs += [
        pl.BlockSpec((TOTAL_DIM, VEC_DIM), lambda i: (0, 0)),
        pl.BlockSpec((TOTAL_DIM, 1), lambda i: (0, 0)),
        pl.BlockSpec(memory_space=pltpu.SMEM),
    ]
    return pl.pallas_call(
        _fm_body,
        grid=(_NB,),
        in_specs=in_specs,
        out_specs=pl.BlockSpec((1, _BB), lambda i: (0, i)),
        out_shape=jax.ShapeDtypeStruct((1, BATCH), jnp.float32),
    )(user_t, item_t, *([feat_t] * N_FIELDS), k_mat, w, b2)


def kernel(inputs, user_table, item_table, feat_tables, w, b, k_mat):
    g_user = jnp.take(user_table, inputs[:, 0], axis=0, mode="clip")
    g_item = jnp.take(item_table, inputs[:, 1], axis=0, mode="clip")
    vocab_ids = inputs[:, 2:].T.astype(jnp.int32)
    field_ids = jax.lax.broadcasted_iota(jnp.int32, (N_FIELDS, BATCH), 0)
    starts = jnp.stack([field_ids, vocab_ids], axis=-1).reshape(-1, 2)
    dnums = lax.GatherDimensionNumbers(
        offset_dims=(1,),
        collapsed_slice_dims=(0, 1),
        start_index_map=(0, 1),
    )
    g_feat = lax.gather(
        feat_tables,
        starts,
        dnums,
        slice_sizes=(1, 1, VEC_DIM),
        mode=lax.GatherScatterMode.PROMISE_IN_BOUNDS,
    )
    yt = _fm(
        g_user.T, g_item.T, g_feat.T, k_mat, w, b.reshape(1, 1)
    )
    return yt.reshape(BATCH, 1)
